# baseline (speedup 1.0000x reference)
import functools

import jax
import jax.numpy as jnp
from jax import lax
from jax.experimental import pallas as pl
from jax.experimental.pallas import tpu as pltpu, tpu_sc as plsc
from jax._src.pallas import mpmd as _mpmd

_EMB = 1000
_BATCH = 16384
_NC = 2
_NS = 16
_NW = _NC * _NS
_JT = _EMB // 8
_IT = _BATCH // 128
_COLS_PER_W = _BATCH // _NW
_FLAT = _JT * _IT * 8 * 128

_mesh = plsc.VectorSubcoreMesh(core_axis_name="c", subcore_axis_name="s")


def _zero_body(o_ref):
    o_ref[...] = jnp.zeros_like(o_ref)


_tc_zero = pl.pallas_call(
    _zero_body,
    out_shape=jax.ShapeDtypeStruct((_JT, _IT, 8, 128), jnp.float32),
    grid=(_IT // 4,),
    out_specs=pl.BlockSpec((_JT, 4, 8, 128), lambda i: (0, i, 0, 0)),
)


def _scatter_body(zeros_hbm, labels_hbm, out_hbm, lab_v, idx_v, ones_v, sem):
    del zeros_hbm
    wid = lax.axis_index("s") * _NC + lax.axis_index("c")
    col0 = wid * _COLS_PER_W

    pltpu.sync_copy(labels_hbm.at[pl.ds(col0, _COLS_PER_W)], lab_v)

    ones = jnp.ones((16,), jnp.float32)
    iota16 = lax.broadcasted_iota(jnp.int32, (16,), 0)
    for jj in range(128 // 16):
        ones_v[pl.ds(jj * 16, 16)] = ones

    def _grp(g, carry):
        j = g >> 3
        lab16 = lab_v[pl.ds(g * 16, 16)]
        i16 = col0 + g * 16 + iota16
        flat16 = (
            (lab16 >> 3) * (_IT * 8 * 128)
            + (i16 >> 7) * (8 * 128)
            + (lab16 & 7) * 128
            + (i16 & 127)
        )
        idx_v[j, pl.ds((g & 7) * 16, 16)] = flat16
        return carry

    lax.fori_loop(0, _COLS_PER_W // 16, _grp, None)

    for j in range(_COLS_PER_W // 128):
        pltpu.async_copy(ones_v, out_hbm.at[idx_v.at[j]], sem)
    for j in range(_COLS_PER_W // 128):
        pltpu.make_async_copy(ones_v, out_hbm.at[idx_v.at[j]], sem).wait()


_sc_scatter = _mpmd._mpmd_map(
    [(_mesh, _scatter_body)],
    [jax.ShapeDtypeStruct((_FLAT,), jnp.float32)],
    input_output_aliases={0: 0},
    compiler_params=pltpu.CompilerParams(needs_layout_passes=False),
    scratch_types=[
        pltpu.VMEM((_COLS_PER_W,), jnp.int32),
        pltpu.VMEM((4, 128), jnp.int32),
        pltpu.VMEM((128,), jnp.float32),
        pltpu.SemaphoreType.DMA,
    ],
)


def kernel(labels):
    z = _tc_zero()
    y = _sc_scatter(z.reshape(_FLAT), labels)[0]
    tiles = y.reshape(_JT, _IT, 8, 128)
    return tiles.transpose((1, 3, 0, 2)).reshape(_BATCH, _EMB)


# speedup vs baseline: 1.3592x; 1.3592x over previous
import functools

import jax
import jax.numpy as jnp
from jax import lax
from jax.experimental import pallas as pl
from jax.experimental.pallas import tpu as pltpu, tpu_sc as plsc

_EMB = 1000
_BATCH = 16384
_NC = 2
_NS = 16
_NW = _NC * _NS
_JT = _EMB // 8
_IT = _BATCH // 128
_TILES_PER_W = _IT // _NW
_COLS_PER_W = _BATCH // _NW
_BANDS = (42, 42, 41)
_J0 = (0, 42, 84)

_mesh = plsc.VectorSubcoreMesh(core_axis_name="c", subcore_axis_name="s")


def _one_hot_body(labels_hbm, out_hbm, lab_v, b0, b1, b2, s0, s1, s2):
    bufs = (b0, b1, b2)
    sems = (s0, s1, s2)
    wid = lax.axis_index("s") * _NC + lax.axis_index("c")
    col0 = wid * _COLS_PER_W
    i_base = wid * _TILES_PER_W

    lab_cp = pltpu.make_async_copy(
        labels_hbm.at[pl.ds(col0, _COLS_PER_W)], lab_v, s0
    )
    lab_cp.start()

    zeros = jnp.zeros((16,), jnp.float32)
    ones = jnp.ones((16,), jnp.float32)
    zeros_i = jnp.zeros((16,), jnp.int32)
    iota16 = lax.broadcasted_iota(jnp.int32, (16,), 0)

    def _zero_buf(buf, nj):
        def _row(j, carry):
            for jr in range(8):
                for jj in range(8):
                    buf[j, 0, jr, pl.ds(jj * 16, 16)] = zeros
            return carry

        lax.fori_loop(0, nj, _row, None)

    def _scatter(b, c, vals):
        lo = _J0[b] * 8
        hi = lo + _BANDS[b] * 8
        buf = bufs[b]

        def _grp(g, carry):
            lab16 = lab_v[pl.ds(c * 128 + g * 16, 16)]
            rel = lab16 - lo
            mask = (lab16 >= lo) & (lab16 < hi)
            plsc.store_scatter(
                buf,
                [rel >> 3, zeros_i, rel & 7, g * 16 + iota16],
                vals,
                mask=mask,
            )
            return carry

        lax.fori_loop(0, 128 // 16, _grp, None)

    def _dma(b, c):
        return pltpu.make_async_copy(
            bufs[b],
            out_hbm.at[pl.ds(_J0[b], _BANDS[b]), pl.ds(i_base + c, 1)],
            sems[b],
        )

    _zero_buf(bufs[0], _BANDS[0])
    lab_cp.wait()
    _scatter(0, 0, ones)
    _dma(0, 0).start()
    for b in (1, 2):
        _zero_buf(bufs[b], _BANDS[b])
        _scatter(b, 0, ones)
        _dma(b, 0).start()

    def _tile(c, carry):
        for b in range(3):
            _dma(b, c).wait()
            _scatter(b, c - 1, zeros)
            _scatter(b, c, ones)
            _dma(b, c).start()
        return carry

    lax.fori_loop(1, _TILES_PER_W, _tile, None)

    for b in range(3):
        _dma(b, 0).wait()


_one_hot_sc = functools.partial(
    pl.kernel,
    out_type=jax.ShapeDtypeStruct((_JT, _IT, 8, 128), jnp.float32),
    mesh=_mesh,
    compiler_params=pltpu.CompilerParams(needs_layout_passes=False),
    scratch_types=[
        pltpu.VMEM((_COLS_PER_W,), jnp.int32),
        pltpu.VMEM((_BANDS[0], 1, 8, 128), jnp.float32),
        pltpu.VMEM((_BANDS[1], 1, 8, 128), jnp.float32),
        pltpu.VMEM((_BANDS[2], 1, 8, 128), jnp.float32),
        pltpu.SemaphoreType.DMA,
        pltpu.SemaphoreType.DMA,
        pltpu.SemaphoreType.DMA,
    ],
)(_one_hot_body)


def kernel(labels):
    tiles = _one_hot_sc(labels)
    return tiles.transpose((1, 3, 0, 2)).reshape(_BATCH, _EMB)
